# baseline (speedup 1.0000x reference)
import jax
import jax.numpy as jnp
from jax.experimental import pallas as pl
from jax.experimental.pallas import tpu as pltpu

HIDDEN = 256
EPS = 1e-6


def _block_kernel(x_ref, p1_ref, p2_ref, w_ref, b_ref, g1_ref, b1_ref,
                  g2_ref, b2_ref, o_ref):
    x = x_ref[0]
    inv_h = 1.0 / x.shape[-1]
    half = p1_ref.shape[1]

    s1 = jnp.sum(x, axis=-1, keepdims=True)
    s2 = jnp.sum(x * x, axis=-1, keepdims=True)
    mu = s1 * inv_h
    r = jax.lax.rsqrt(s2 * inv_h - mu * mu + EPS)
    h = ((x - mu) * r) * g1_ref[0] + b1_ref[0]

    for j, p_ref in ((0, p1_ref), (1, p2_ref)):
        p = p_ref[0]
        sl = slice(j * half, (j + 1) * half)
        agg = jnp.dot(p, h, preferred_element_type=jnp.float32)
        conv = jnp.maximum(
            jnp.dot(agg, w_ref[...], preferred_element_type=jnp.float32)
            + b_ref[0], 0.0)
        y = x[sl] + conv
        mu2 = jnp.sum(y, axis=-1, keepdims=True) * inv_h
        n2 = jnp.sum(y * y, axis=-1, keepdims=True) * inv_h
        r2 = jax.lax.rsqrt(n2 - mu2 * mu2 + EPS)
        o_ref[0, sl] = ((y - mu2) * r2) * g2_ref[0] + b2_ref[0]


def kernel(x, mask, inputP, W, b, ln1_g, ln1_b, ln2_g, ln2_b):
    del mask
    B, N, H = x.shape
    HN = N // 2

    vec = lambda v: v.reshape(1, H)
    return pl.pallas_call(
        _block_kernel,
        grid=(B,),
        in_specs=[
            pl.BlockSpec((1, N, H), lambda i: (i, 0, 0)),
            pl.BlockSpec((1, HN, N), lambda i: (i, 0, 0)),
            pl.BlockSpec((1, HN, N), lambda i: (i, 1, 0)),
            pl.BlockSpec((H, H), lambda i: (0, 0)),
            pl.BlockSpec((1, H), lambda i: (0, 0)),
            pl.BlockSpec((1, H), lambda i: (0, 0)),
            pl.BlockSpec((1, H), lambda i: (0, 0)),
            pl.BlockSpec((1, H), lambda i: (0, 0)),
            pl.BlockSpec((1, H), lambda i: (0, 0)),
        ],
        out_specs=pl.BlockSpec((1, N, H), lambda i: (i, 0, 0)),
        out_shape=jax.ShapeDtypeStruct((B, N, H), x.dtype),
        compiler_params=pltpu.CompilerParams(
            dimension_semantics=("arbitrary",)),
    )(x, inputP, inputP, W, vec(b), vec(ln1_g), vec(ln1_b), vec(ln2_g),
      vec(ln2_b))


# speedup vs baseline: 1.1017x; 1.1017x over previous
import jax
import jax.numpy as jnp
from jax.experimental import pallas as pl
from jax.experimental.pallas import tpu as pltpu

HIDDEN = 256
EPS = 1e-6


def _block_kernel(x_ref, p_ref, w_ref, b_ref, g1_ref, b1_ref, g2_ref, b2_ref,
                  o_ref):
    x = x_ref[0]
    p = p_ref[0]
    inv_h = 1.0 / x.shape[-1]

    s1 = jnp.sum(x, axis=-1, keepdims=True)
    s2 = jnp.sum(x * x, axis=-1, keepdims=True)
    mu = s1 * inv_h
    r = jax.lax.rsqrt(s2 * inv_h - mu * mu + EPS)
    h = ((x - mu) * r) * g1_ref[0] + b1_ref[0]

    agg = jnp.dot(p, h, preferred_element_type=jnp.float32)
    conv = jnp.maximum(
        jnp.dot(agg, w_ref[...], preferred_element_type=jnp.float32)
        + b_ref[0], 0.0)

    y = x + conv
    mu2 = jnp.sum(y, axis=-1, keepdims=True) * inv_h
    n2 = jnp.sum(y * y, axis=-1, keepdims=True) * inv_h
    r2 = jax.lax.rsqrt(n2 - mu2 * mu2 + EPS)
    o_ref[0] = ((y - mu2) * r2) * g2_ref[0] + b2_ref[0]


def kernel(x, mask, inputP, W, b, ln1_g, ln1_b, ln2_g, ln2_b):
    del mask
    B, N, H = x.shape

    vec = lambda v: v.reshape(1, H)
    return pl.pallas_call(
        _block_kernel,
        grid=(B,),
        in_specs=[
            pl.BlockSpec((1, N, H), lambda i: (i, 0, 0)),
            pl.BlockSpec((1, N, N), lambda i: (i, 0, 0)),
            pl.BlockSpec((H, H), lambda i: (0, 0)),
            pl.BlockSpec((1, H), lambda i: (0, 0)),
            pl.BlockSpec((1, H), lambda i: (0, 0)),
            pl.BlockSpec((1, H), lambda i: (0, 0)),
            pl.BlockSpec((1, H), lambda i: (0, 0)),
            pl.BlockSpec((1, H), lambda i: (0, 0)),
        ],
        out_specs=pl.BlockSpec((1, N, H), lambda i: (i, 0, 0)),
        out_shape=jax.ShapeDtypeStruct((B, N, H), x.dtype),
        compiler_params=pltpu.CompilerParams(
            dimension_semantics=("arbitrary",)),
    )(x, inputP, W, vec(b), vec(ln1_g), vec(ln1_b), vec(ln2_g), vec(ln2_b))
